# baseline (speedup 1.0000x reference)
import dataclasses
import functools

import jax
import jax.numpy as jnp
from jax import lax
from jax.experimental import pallas as pl
from jax.experimental.pallas import tpu as pltpu
from jax.experimental.pallas import tpu_sc as plsc
from jax._src.pallas import core as _pallas_core
from jax._src.pallas import mpmd as plmpmd
from jax._src.pallas.mosaic import core as _tpu_core

_NC, _NS = 2, 16
_NW = _NC * _NS
_SEQ = 8192
_DIM = 2048

_TEC_ROWS = 6144
_T_ROWS_PER_W = _TEC_ROWS // _NW
_T_CHUNK = 16
_T_NCHUNKS = _T_ROWS_PER_W // _T_CHUNK
_T_NBUF = 2

_SCS_ROWS = _SEQ - _TEC_ROWS
_S_ROWS_PER_C = _SCS_ROWS // _NC
_S_CHUNK = 256
_S_NCHUNKS = _S_ROWS_PER_C // _S_CHUNK
_S_NBUF = 2


def _pipeline(cp_in, cp_out, nchunks, nbuf):
    for c in range(nbuf):
        cp_in(c).start()
    for c in range(nchunks):
        cp_in(c).wait()
        cp_out(c).start()
        if c + nbuf < nchunks:
            cp_out(c).wait()
            cp_in(c + nbuf).start()
    for c in range(max(0, nchunks - nbuf), nchunks):
        cp_out(c).wait()


def _tec_fn(w_hbm, out_hbm, t_buf, t_sems, s_buf, s_sems):
    del s_buf, s_sems
    wid = lax.axis_index("s") * _NC + lax.axis_index("c")
    base = wid * _T_ROWS_PER_W

    def cp_in(c):
        return pltpu.make_async_copy(
            w_hbm.at[pl.ds(base + c * _T_CHUNK, _T_CHUNK), :],
            t_buf.at[c % _T_NBUF],
            t_sems.at[c % _T_NBUF],
        )

    def cp_out(c):
        return pltpu.make_async_copy(
            t_buf.at[c % _T_NBUF],
            out_hbm.at[pl.ds(base + c * _T_CHUNK, _T_CHUNK), :],
            t_sems.at[_T_NBUF + c % _T_NBUF],
        )

    _pipeline(cp_in, cp_out, _T_NCHUNKS, _T_NBUF)


def _scs_fn(w_hbm, out_hbm, t_buf, t_sems, s_buf, s_sems):
    del t_buf, t_sems
    cid = lax.axis_index("c")
    base = _TEC_ROWS + cid * _S_ROWS_PER_C

    def cp_in(c):
        return pltpu.make_async_copy(
            w_hbm.at[pl.ds(base + c * _S_CHUNK, _S_CHUNK), :],
            s_buf.at[c % _S_NBUF],
            s_sems.at[c % _S_NBUF],
        )

    def cp_out(c):
        return pltpu.make_async_copy(
            s_buf.at[c % _S_NBUF],
            out_hbm.at[pl.ds(base + c * _S_CHUNK, _S_CHUNK), :],
            s_sems.at[_S_NBUF + c % _S_NBUF],
        )

    _pipeline(cp_in, cp_out, _S_NCHUNKS, _S_NBUF)


def _core_typed(mem_ref, memory_space, mesh):
    return dataclasses.replace(
        mem_ref,
        memory_space=_pallas_core.CoreMemorySpace(memory_space, mesh),
    )


def kernel(x, emb_weight):
    seq_len = x.shape[1]
    dim = emb_weight.shape[1]
    scalar_mesh = plsc.ScalarSubcoreMesh(axis_name="c", num_cores=_NC)
    vector_mesh = plsc.VectorSubcoreMesh(core_axis_name="c", subcore_axis_name="s")
    sem_t = pltpu.SemaphoreType.DMA((2 * _T_NBUF,))
    sem_s = pltpu.SemaphoreType.DMA((2 * _S_NBUF,))
    k = plmpmd.mpmd_map(
        [(scalar_mesh, _scs_fn), (vector_mesh, _tec_fn)],
        out_types=jax.ShapeDtypeStruct((seq_len, dim), emb_weight.dtype),
        scratch_types=[
            _pallas_core.CoreMemorySpace(_tpu_core.MemorySpace.VMEM, vector_mesh)(
                (_T_NBUF, _T_CHUNK, dim), jnp.float32
            ),
            _core_typed(sem_t, _tpu_core.MemorySpace.SEMAPHORE, vector_mesh),
            pltpu.VMEM_SHARED((_S_NBUF, _S_CHUNK, dim), jnp.float32),
            _core_typed(sem_s, _tpu_core.MemorySpace.SEMAPHORE, scalar_mesh),
        ],
    )
    out = k(emb_weight[:seq_len])
    return out[None]


# speedup vs baseline: 1.0059x; 1.0059x over previous
import dataclasses
import functools

import jax
import jax.numpy as jnp
from jax import lax
from jax.experimental import pallas as pl
from jax.experimental.pallas import tpu as pltpu
from jax.experimental.pallas import tpu_sc as plsc
from jax._src.pallas import core as _pallas_core
from jax._src.pallas import mpmd as plmpmd
from jax._src.pallas.mosaic import core as _tpu_core

_NC, _NS = 2, 16
_NW = _NC * _NS
_SEQ = 8192
_DIM = 2048

_TEC_ROWS = 5632
_T_ROWS_PER_W = _TEC_ROWS // _NW
_T_CHUNK = 16
_T_NCHUNKS = _T_ROWS_PER_W // _T_CHUNK
_T_NBUF = 2

_SCS_ROWS = _SEQ - _TEC_ROWS
_S_ROWS_PER_C = _SCS_ROWS // _NC
_S_CHUNK = 256
_S_NCHUNKS = _S_ROWS_PER_C // _S_CHUNK
_S_NBUF = 2


def _pipeline(cp_in, cp_out, nchunks, nbuf):
    for c in range(nbuf):
        cp_in(c).start()
    for c in range(nchunks):
        cp_in(c).wait()
        cp_out(c).start()
        if c + nbuf < nchunks:
            cp_out(c).wait()
            cp_in(c + nbuf).start()
    for c in range(max(0, nchunks - nbuf), nchunks):
        cp_out(c).wait()


def _tec_fn(w_hbm, out_hbm, t_buf, t_sems, s_buf, s_sems):
    del s_buf, s_sems
    wid = lax.axis_index("s") * _NC + lax.axis_index("c")
    base = wid * _T_ROWS_PER_W

    def cp_in(c):
        return pltpu.make_async_copy(
            w_hbm.at[pl.ds(base + c * _T_CHUNK, _T_CHUNK), :],
            t_buf.at[c % _T_NBUF],
            t_sems.at[c % _T_NBUF],
        )

    def cp_out(c):
        return pltpu.make_async_copy(
            t_buf.at[c % _T_NBUF],
            out_hbm.at[pl.ds(base + c * _T_CHUNK, _T_CHUNK), :],
            t_sems.at[_T_NBUF + c % _T_NBUF],
        )

    _pipeline(cp_in, cp_out, _T_NCHUNKS, _T_NBUF)


def _scs_fn(w_hbm, out_hbm, t_buf, t_sems, s_buf, s_sems):
    del t_buf, t_sems
    cid = lax.axis_index("c")
    base = _TEC_ROWS + cid * _S_ROWS_PER_C

    def cp_in(c):
        return pltpu.make_async_copy(
            w_hbm.at[pl.ds(base + c * _S_CHUNK, _S_CHUNK), :],
            s_buf.at[c % _S_NBUF],
            s_sems.at[c % _S_NBUF],
        )

    def cp_out(c):
        return pltpu.make_async_copy(
            s_buf.at[c % _S_NBUF],
            out_hbm.at[pl.ds(base + c * _S_CHUNK, _S_CHUNK), :],
            s_sems.at[_S_NBUF + c % _S_NBUF],
        )

    _pipeline(cp_in, cp_out, _S_NCHUNKS, _S_NBUF)


def _core_typed(mem_ref, memory_space, mesh):
    return dataclasses.replace(
        mem_ref,
        memory_space=_pallas_core.CoreMemorySpace(memory_space, mesh),
    )


def kernel(x, emb_weight):
    seq_len = x.shape[1]
    dim = emb_weight.shape[1]
    scalar_mesh = plsc.ScalarSubcoreMesh(axis_name="c", num_cores=_NC)
    vector_mesh = plsc.VectorSubcoreMesh(core_axis_name="c", subcore_axis_name="s")
    sem_t = pltpu.SemaphoreType.DMA((2 * _T_NBUF,))
    sem_s = pltpu.SemaphoreType.DMA((2 * _S_NBUF,))
    k = plmpmd.mpmd_map(
        [(scalar_mesh, _scs_fn), (vector_mesh, _tec_fn)],
        out_types=jax.ShapeDtypeStruct((seq_len, dim), emb_weight.dtype),
        scratch_types=[
            _pallas_core.CoreMemorySpace(_tpu_core.MemorySpace.VMEM, vector_mesh)(
                (_T_NBUF, _T_CHUNK, dim), jnp.float32
            ),
            _core_typed(sem_t, _tpu_core.MemorySpace.SEMAPHORE, vector_mesh),
            pltpu.VMEM_SHARED((_S_NBUF, _S_CHUNK, dim), jnp.float32),
            _core_typed(sem_s, _tpu_core.MemorySpace.SEMAPHORE, scalar_mesh),
        ],
    )
    out = k(emb_weight[:seq_len])
    return out[None]


# speedup vs baseline: 1.0100x; 1.0040x over previous
import dataclasses

import jax
import jax.numpy as jnp
from jax import lax
from jax.experimental import pallas as pl
from jax.experimental.pallas import tpu as pltpu
from jax.experimental.pallas import tpu_sc as plsc
from jax._src.pallas import core as _pallas_core
from jax._src.pallas import mpmd as plmpmd
from jax._src.pallas.mosaic import core as _tpu_core

_NC, _NS = 2, 16
_NW = _NC * _NS
_SEQ = 8192
_DIM = 2048

_TEC_ROWS = 5120
_T_ROWS_PER_W = _TEC_ROWS // _NW
_T_CHUNK = 16
_T_NCHUNKS = _T_ROWS_PER_W // _T_CHUNK
_T_NBUF = 2

_SCS_ROWS = _SEQ - _TEC_ROWS
_S_ROWS_PER_C = _SCS_ROWS // _NC
_S_CHUNK = 256
_S_NCHUNKS = _S_ROWS_PER_C // _S_CHUNK
_S_NBUF = 2


def _pipeline(cp_in, cp_out, nchunks, nbuf):
    for c in range(nbuf):
        cp_in(c).start()
    for c in range(nchunks):
        cp_in(c).wait()
        cp_out(c).start()
        if c + nbuf < nchunks:
            cp_out(c).wait()
            cp_in(c + nbuf).start()
    for c in range(max(0, nchunks - nbuf), nchunks):
        cp_out(c).wait()


def _tec_fn(w_hbm, out_hbm, t_buf, t_sems, s_buf, s_sems):
    del s_buf, s_sems
    wid = lax.axis_index("s") * _NC + lax.axis_index("c")
    base = wid * _T_ROWS_PER_W

    def cp_in(c):
        return pltpu.make_async_copy(
            w_hbm.at[pl.ds(base + c * _T_CHUNK, _T_CHUNK), :],
            t_buf.at[c % _T_NBUF],
            t_sems.at[c % _T_NBUF],
        )

    def cp_out(c):
        return pltpu.make_async_copy(
            t_buf.at[c % _T_NBUF],
            out_hbm.at[pl.ds(base + c * _T_CHUNK, _T_CHUNK), :],
            t_sems.at[_T_NBUF + c % _T_NBUF],
        )

    _pipeline(cp_in, cp_out, _T_NCHUNKS, _T_NBUF)


def _scs_fn(w_hbm, out_hbm, t_buf, t_sems, s_buf, s_sems):
    del t_buf, t_sems
    cid = lax.axis_index("c")
    base = _TEC_ROWS + cid * _S_ROWS_PER_C

    def cp_in(c):
        return pltpu.make_async_copy(
            w_hbm.at[pl.ds(base + c * _S_CHUNK, _S_CHUNK), :],
            s_buf.at[c % _S_NBUF],
            s_sems.at[c % _S_NBUF],
        )

    def cp_out(c):
        return pltpu.make_async_copy(
            s_buf.at[c % _S_NBUF],
            out_hbm.at[pl.ds(base + c * _S_CHUNK, _S_CHUNK), :],
            s_sems.at[_S_NBUF + c % _S_NBUF],
        )

    _pipeline(cp_in, cp_out, _S_NCHUNKS, _S_NBUF)


def _core_typed(mem_ref, memory_space, mesh):
    return dataclasses.replace(
        mem_ref,
        memory_space=_pallas_core.CoreMemorySpace(memory_space, mesh),
    )


def kernel(x, emb_weight):
    seq_len = x.shape[1]
    dim = emb_weight.shape[1]
    scalar_mesh = plsc.ScalarSubcoreMesh(axis_name="c", num_cores=_NC)
    vector_mesh = plsc.VectorSubcoreMesh(core_axis_name="c", subcore_axis_name="s")
    sem_t = pltpu.SemaphoreType.DMA((2 * _T_NBUF,))
    sem_s = pltpu.SemaphoreType.DMA((2 * _S_NBUF,))
    k = plmpmd.mpmd_map(
        [(scalar_mesh, _scs_fn), (vector_mesh, _tec_fn)],
        out_types=jax.ShapeDtypeStruct((seq_len, dim), emb_weight.dtype),
        scratch_types=[
            _pallas_core.CoreMemorySpace(_tpu_core.MemorySpace.VMEM, vector_mesh)(
                (_T_NBUF, _T_CHUNK, dim), jnp.float32
            ),
            _core_typed(sem_t, _tpu_core.MemorySpace.SEMAPHORE, vector_mesh),
            pltpu.VMEM_SHARED((_S_NBUF, _S_CHUNK, dim), jnp.float32),
            _core_typed(sem_s, _tpu_core.MemorySpace.SEMAPHORE, scalar_mesh),
        ],
    )
    out = k(emb_weight[:seq_len])
    return out[None]
